# baseline (speedup 1.0000x reference)
import functools

import jax
import jax.numpy as jnp
from jax import lax
from jax.experimental import pallas as pl
from jax.experimental.pallas import tpu as pltpu
from jax.experimental.pallas import tpu_sc as plsc

NC = 2
NS = 16
L = 16
NW = NC * NS

BATCH = 16384
D = 64
B_PER_W = BATCH // NW
CHUNK = 64
NCHUNK = B_PER_W // CHUNK
NR = 1000
MARGIN = 1.0


NE = 100000
NBLK = (NE + 127) // 128


TCW = 8192
NTC = (NE + TCW - 1) // TCW


def _tc_transpose_body(in_ref, out_ref):
  x = in_ref[...].T
  pieces = []
  for s in range(TCW // 128):
    pieces.append(jnp.concatenate(
        [x[128 * s:128 * s + 64, :], x[128 * s + 64:128 * s + 128, :]],
        axis=1))
  out_ref[...] = jnp.concatenate(pieces, axis=0)


def _transe_body(ent_hbm, rel_hbm, hp_hbm, rp_hbm, tp_hbm, hn_hbm, rn_hbm,
                 tn_hbm, out_hbm, idx_v, ridx_v, row_v, rel_v, out_v, sem):
  wid = lax.axis_index("s") * NC + lax.axis_index("c")
  pltpu.sync_copy(rel_hbm, rel_v)

  def fire(c, buf):
    for a, src in enumerate((hp_hbm, tp_hbm, hn_hbm, tn_hbm)):
      pltpu.sync_copy(src.at[wid, c], idx_v.at[buf, a])
    for a, src in enumerate((rp_hbm, rn_hbm)):
      pltpu.sync_copy(src.at[wid, c], ridx_v.at[buf, a, pl.ds(0, CHUNK)])
    for a in range(4):
      for j in range(CHUNK // L):
        sl = pl.ds(j * L, L)
        e = idx_v[buf, a, sl]
        idx_v[buf, a, sl] = ((e >> 7) << 7) + ((e & 63) << 1) + ((e >> 6) & 1)
    for a in range(4):
      pltpu.make_async_copy(ent_hbm.at[idx_v.at[buf, a]], row_v.at[buf, a],
                            sem.at[buf]).start()

  def drain(buf):
    for a in range(4):
      pltpu.make_async_copy(ent_hbm.at[idx_v.at[buf, a]], row_v.at[buf, a],
                            sem.at[buf]).wait()

  lane = lax.iota(jnp.int32, L)

  def compute(buf, total):
    def body(t, tot):
      rp = ridx_v[buf, 0, pl.ds(t, L)][0]
      rn = ridx_v[buf, 1, pl.ds(t, L)][0]
      dv = jnp.zeros((L,), jnp.float32)
      for k in range(D // L):
        sl = pl.ds(k * L, L)
        dv += jnp.abs(row_v[buf, 0, t, sl] + rel_v[rp, sl]
                      - row_v[buf, 1, t, sl])
        dv -= jnp.abs(row_v[buf, 2, t, sl] + rel_v[rn, sl]
                      - row_v[buf, 3, t, sl])
      for shift in (1, 2, 4, 8):
        dv = dv + jnp.take_along_axis(dv, lane ^ shift, axis=0,
                                      mode="promise_in_bounds")
      return tot + jnp.maximum(dv + MARGIN, 0.0)

    return lax.fori_loop(0, CHUNK, body, total)

  total = jnp.zeros((L,), jnp.float32)
  fire(0, 0)
  for c in range(1, NCHUNK):
    fire(c, c % 2)
    drain((c - 1) % 2)
    total = compute((c - 1) % 2, total)
  drain((NCHUNK - 1) % 2)
  total = compute((NCHUNK - 1) % 2, total)

  out_v[...] = jnp.where(lane == 0, total, 0.0)
  pltpu.sync_copy(out_v, out_hbm.at[wid])


@jax.jit
def _transe_call(entity_emb, relation_emb, hp, rp, tp, hn, rn, tn):
  mesh = plsc.VectorSubcoreMesh(
      core_axis_name="c", subcore_axis_name="s", num_cores=NC,
      num_subcores=NS)
  transpose_kernel = pl.pallas_call(
      _tc_transpose_body,
      out_shape=jax.ShapeDtypeStruct((NTC * TCW // 2, 128), jnp.float32),
      grid=(NTC,),
      in_specs=[pl.BlockSpec((D, TCW), lambda i: (0, i))],
      out_specs=pl.BlockSpec((TCW // 2, 128), lambda i: (i, 0)),
  )
  ent_lin = transpose_kernel(entity_emb.T).reshape(NTC * TCW, D)
  grid_kernel = pl.kernel(
      _transe_body,
      out_type=jax.ShapeDtypeStruct((NW, L), jnp.float32),
      mesh=mesh,
      scratch_types=[
          pltpu.VMEM((2, 4, CHUNK), jnp.int32),
          pltpu.VMEM((2, 2, CHUNK + L), jnp.int32),
          pltpu.VMEM((2, 4, CHUNK, D), jnp.float32),
          pltpu.VMEM((NR, D), jnp.float32),
          pltpu.VMEM((L,), jnp.float32),
          pltpu.SemaphoreType.DMA((2,)),
      ],
      compiler_params=pltpu.CompilerParams(use_tc_tiling_on_sc=False),
  )
  partials = grid_kernel(ent_lin, relation_emb, hp, rp, tp, hn, rn, tn)
  return jnp.sum(partials) / BATCH


def kernel(entity_emb, relation_emb, h_pos, r_pos, t_pos, h_neg, r_neg,
           t_neg):
  shape = (NW, NCHUNK, CHUNK)
  return _transe_call(
      entity_emb, relation_emb,
      h_pos.astype(jnp.int32).reshape(shape),
      r_pos.astype(jnp.int32).reshape(shape),
      t_pos.astype(jnp.int32).reshape(shape),
      h_neg.astype(jnp.int32).reshape(shape),
      r_neg.astype(jnp.int32).reshape(shape),
      t_neg.astype(jnp.int32).reshape(shape),
  )


# speedup vs baseline: 1.2233x; 1.2233x over previous
import functools

import jax
import jax.numpy as jnp
from jax import lax
from jax.experimental import pallas as pl
from jax.experimental.pallas import tpu as pltpu
from jax.experimental.pallas import tpu_sc as plsc

NC = 2
NS = 16
L = 16
NW = NC * NS

BATCH = 16384
D = 64
B_PER_W = BATCH // NW
CHUNK = 128
NCHUNK = B_PER_W // CHUNK
MARGIN = 1.0


NE = 100000
NBLK = (NE + 127) // 128


TCW = 8192
NTC = (NE + TCW - 1) // TCW


def _tc_transpose_body(in_ref, out_ref):
  x = in_ref[...].T
  pieces = []
  for s in range(TCW // 128):
    pieces.append(jnp.concatenate(
        [x[128 * s:128 * s + 64, :], x[128 * s + 64:128 * s + 128, :]],
        axis=1))
  out_ref[...] = jnp.concatenate(pieces, axis=0)


def _transe_body(ent_hbm, rel_hbm, hp_hbm, rp_hbm, tp_hbm, hn_hbm, rn_hbm,
                 tn_hbm, out_hbm, idx_v, row_v, out_v, sem):
  wid = lax.axis_index("s") * NC + lax.axis_index("c")

  def fire(c, buf):
    for a, src in enumerate((hp_hbm, rp_hbm, tp_hbm, hn_hbm, rn_hbm,
                             tn_hbm)):
      pltpu.sync_copy(src.at[wid, c], idx_v.at[buf, a])
    for a in (0, 2, 3, 5):
      for j in range(CHUNK // L):
        sl = pl.ds(j * L, L)
        e = idx_v[buf, a, sl]
        idx_v[buf, a, sl] = ((e >> 7) << 7) + ((e & 63) << 1) + ((e >> 6) & 1)
    for a, table in enumerate((ent_hbm, rel_hbm, ent_hbm, ent_hbm, rel_hbm,
                               ent_hbm)):
      pltpu.make_async_copy(table.at[idx_v.at[buf, a]], row_v.at[buf, a],
                            sem.at[buf]).start()

  def drain(buf):
    for a, table in enumerate((ent_hbm, rel_hbm, ent_hbm, ent_hbm, rel_hbm,
                               ent_hbm)):
      pltpu.make_async_copy(table.at[idx_v.at[buf, a]], row_v.at[buf, a],
                            sem.at[buf]).wait()

  lane = lax.iota(jnp.int32, L)

  def compute(buf, total):
    def body(t, tot):
      dv = jnp.zeros((L,), jnp.float32)
      for k in range(D // L):
        sl = pl.ds(k * L, L)
        dv += jnp.abs(row_v[buf, 0, t, sl] + row_v[buf, 1, t, sl]
                      - row_v[buf, 2, t, sl])
        dv -= jnp.abs(row_v[buf, 3, t, sl] + row_v[buf, 4, t, sl]
                      - row_v[buf, 5, t, sl])
      for shift in (1, 2, 4, 8):
        dv = dv + jnp.take_along_axis(dv, lane ^ shift, axis=0,
                                      mode="promise_in_bounds")
      return tot + jnp.maximum(dv + MARGIN, 0.0)

    return lax.fori_loop(0, CHUNK, body, total, unroll=2)

  total = jnp.zeros((L,), jnp.float32)
  fire(0, 0)
  for c in range(1, NCHUNK):
    fire(c, c % 2)
    drain((c - 1) % 2)
    total = compute((c - 1) % 2, total)
  drain((NCHUNK - 1) % 2)
  total = compute((NCHUNK - 1) % 2, total)

  out_v[...] = jnp.where(lane == 0, total, 0.0)
  pltpu.sync_copy(out_v, out_hbm.at[wid])


@jax.jit
def _transe_call(entity_emb, relation_emb, hp, rp, tp, hn, rn, tn):
  mesh = plsc.VectorSubcoreMesh(
      core_axis_name="c", subcore_axis_name="s", num_cores=NC,
      num_subcores=NS)
  transpose_kernel = pl.pallas_call(
      _tc_transpose_body,
      out_shape=jax.ShapeDtypeStruct((NTC * TCW // 2, 128), jnp.float32),
      grid=(NTC,),
      in_specs=[pl.BlockSpec((D, TCW), lambda i: (0, i))],
      out_specs=pl.BlockSpec((TCW // 2, 128), lambda i: (i, 0)),
  )
  ent_lin = transpose_kernel(entity_emb.T).reshape(NTC * TCW, D)
  grid_kernel = pl.kernel(
      _transe_body,
      out_type=jax.ShapeDtypeStruct((NW, L), jnp.float32),
      mesh=mesh,
      scratch_types=[
          pltpu.VMEM((2, 6, CHUNK), jnp.int32),
          pltpu.VMEM((2, 6, CHUNK, D), jnp.float32),
          pltpu.VMEM((L,), jnp.float32),
          pltpu.SemaphoreType.DMA((2,)),
      ],
      compiler_params=pltpu.CompilerParams(use_tc_tiling_on_sc=False),
  )
  partials = grid_kernel(ent_lin, relation_emb, hp, rp, tp, hn, rn, tn)
  return jnp.sum(partials) / BATCH


def kernel(entity_emb, relation_emb, h_pos, r_pos, t_pos, h_neg, r_neg,
           t_neg):
  shape = (NW, NCHUNK, CHUNK)
  return _transe_call(
      entity_emb, relation_emb,
      h_pos.astype(jnp.int32).reshape(shape),
      r_pos.astype(jnp.int32).reshape(shape),
      t_pos.astype(jnp.int32).reshape(shape),
      h_neg.astype(jnp.int32).reshape(shape),
      r_neg.astype(jnp.int32).reshape(shape),
      t_neg.astype(jnp.int32).reshape(shape),
  )


# speedup vs baseline: 1.2601x; 1.0301x over previous
import functools

import jax
import jax.numpy as jnp
from jax import lax
from jax.experimental import pallas as pl
from jax.experimental.pallas import tpu as pltpu
from jax.experimental.pallas import tpu_sc as plsc

NC = 2
NS = 16
L = 16
NW = NC * NS

BATCH = 16384
D = 64
B_PER_W = BATCH // NW
CHUNK = 128
NCHUNK = B_PER_W // CHUNK
MARGIN = 1.0


NE = 100000
NBLK = (NE + 127) // 128


TCW = 12800
NTC = (NE + TCW - 1) // TCW


def _tc_transpose_body(in_ref, out_ref):
  x = in_ref[...].T
  pieces = []
  for s in range(TCW // 128):
    pieces.append(jnp.concatenate(
        [x[128 * s:128 * s + 64, :], x[128 * s + 64:128 * s + 128, :]],
        axis=1))
  out_ref[...] = jnp.concatenate(pieces, axis=0)


def _transe_body(ent_hbm, rel_hbm, hp_hbm, rp_hbm, tp_hbm, hn_hbm, rn_hbm,
                 tn_hbm, out_hbm, idx_v, row_v, out_v, sem):
  wid = lax.axis_index("s") * NC + lax.axis_index("c")

  def fire(c, buf):
    for a, src in enumerate((hp_hbm, rp_hbm, tp_hbm, hn_hbm, rn_hbm,
                             tn_hbm)):
      pltpu.sync_copy(src.at[wid, c], idx_v.at[buf, a])
    for a in (0, 2, 3, 5):
      for j in range(CHUNK // L):
        sl = pl.ds(j * L, L)
        e = idx_v[buf, a, sl]
        idx_v[buf, a, sl] = ((e >> 7) << 7) + ((e & 63) << 1) + ((e >> 6) & 1)
    for a, table in enumerate((ent_hbm, rel_hbm, ent_hbm, ent_hbm, rel_hbm,
                               ent_hbm)):
      pltpu.make_async_copy(table.at[idx_v.at[buf, a]], row_v.at[buf, a],
                            sem.at[buf]).start()

  def drain(buf):
    for a, table in enumerate((ent_hbm, rel_hbm, ent_hbm, ent_hbm, rel_hbm,
                               ent_hbm)):
      pltpu.make_async_copy(table.at[idx_v.at[buf, a]], row_v.at[buf, a],
                            sem.at[buf]).wait()

  lane = lax.iota(jnp.int32, L)

  def compute(buf, total):
    def body(t, tot):
      dv = jnp.zeros((L,), jnp.float32)
      for k in range(D // L):
        sl = pl.ds(k * L, L)
        dv += jnp.abs(row_v[buf, 0, t, sl] + row_v[buf, 1, t, sl]
                      - row_v[buf, 2, t, sl])
        dv -= jnp.abs(row_v[buf, 3, t, sl] + row_v[buf, 4, t, sl]
                      - row_v[buf, 5, t, sl])
      for shift in (1, 2, 4, 8):
        dv = dv + jnp.take_along_axis(dv, lane ^ shift, axis=0,
                                      mode="promise_in_bounds")
      return tot + jnp.maximum(dv + MARGIN, 0.0)

    return lax.fori_loop(0, CHUNK, body, total, unroll=2)

  total = jnp.zeros((L,), jnp.float32)
  fire(0, 0)
  for c in range(1, NCHUNK):
    fire(c, c % 2)
    drain((c - 1) % 2)
    total = compute((c - 1) % 2, total)
  drain((NCHUNK - 1) % 2)
  total = compute((NCHUNK - 1) % 2, total)

  out_v[...] = jnp.where(lane == 0, total, 0.0)
  pltpu.sync_copy(out_v, out_hbm.at[wid])


@jax.jit
def _transe_call(entity_emb, relation_emb, hp, rp, tp, hn, rn, tn):
  mesh = plsc.VectorSubcoreMesh(
      core_axis_name="c", subcore_axis_name="s", num_cores=NC,
      num_subcores=NS)
  transpose_kernel = pl.pallas_call(
      _tc_transpose_body,
      out_shape=jax.ShapeDtypeStruct((NTC * TCW // 2, 128), jnp.float32),
      grid=(NTC,),
      in_specs=[pl.BlockSpec((D, TCW), lambda i: (0, i))],
      out_specs=pl.BlockSpec((TCW // 2, 128), lambda i: (i, 0)),
  )
  ent_lin = transpose_kernel(entity_emb.T).reshape(NTC * TCW, D)
  grid_kernel = pl.kernel(
      _transe_body,
      out_type=jax.ShapeDtypeStruct((NW, L), jnp.float32),
      mesh=mesh,
      scratch_types=[
          pltpu.VMEM((2, 6, CHUNK), jnp.int32),
          pltpu.VMEM((2, 6, CHUNK, D), jnp.float32),
          pltpu.VMEM((L,), jnp.float32),
          pltpu.SemaphoreType.DMA((2,)),
      ],
      compiler_params=pltpu.CompilerParams(use_tc_tiling_on_sc=False),
  )
  partials = grid_kernel(ent_lin, relation_emb, hp, rp, tp, hn, rn, tn)
  return jnp.sum(partials) / BATCH


def kernel(entity_emb, relation_emb, h_pos, r_pos, t_pos, h_neg, r_neg,
           t_neg):
  shape = (NW, NCHUNK, CHUNK)
  return _transe_call(
      entity_emb, relation_emb,
      h_pos.astype(jnp.int32).reshape(shape),
      r_pos.astype(jnp.int32).reshape(shape),
      t_pos.astype(jnp.int32).reshape(shape),
      h_neg.astype(jnp.int32).reshape(shape),
      r_neg.astype(jnp.int32).reshape(shape),
      t_neg.astype(jnp.int32).reshape(shape),
  )


# speedup vs baseline: 1.2646x; 1.0036x over previous
import jax
import jax.numpy as jnp
from jax import lax
from jax.experimental import pallas as pl
from jax.experimental.pallas import tpu as pltpu
from jax.experimental.pallas import tpu_sc as plsc

NC = 2
NS = 16
L = 16
NW = NC * NS

BATCH = 16384
D = 64
B_PER_W = BATCH // NW
CHUNK = 128
NCHUNK = B_PER_W // CHUNK
MARGIN = 1.0


NE = 100000
TCW = 12800
NTC = (NE + TCW - 1) // TCW


def _tc_transpose_body(in_ref, out_ref):
  x = in_ref[...].T
  pieces = []
  for s in range(TCW // 128):
    pieces.append(jnp.concatenate(
        [x[128 * s:128 * s + 64, :], x[128 * s + 64:128 * s + 128, :]],
        axis=1))
  out_ref[...] = jnp.concatenate(pieces, axis=0)


def _transe_body(ent_hbm, rel_hbm, hp_hbm, rp_hbm, tp_hbm, hn_hbm, rn_hbm,
                 tn_hbm, out_hbm, idx_v, row_v, out_v, sem):
  wid = lax.axis_index("s") * NC + lax.axis_index("c")

  def fire(c, buf):
    for a, src in enumerate((hp_hbm, rp_hbm, tp_hbm, hn_hbm, rn_hbm,
                             tn_hbm)):
      pltpu.sync_copy(src.at[wid, c], idx_v.at[buf, a])
    for a in (0, 2, 3, 5):
      for j in range(CHUNK // L):
        sl = pl.ds(j * L, L)
        e = idx_v[buf, a, sl]
        idx_v[buf, a, sl] = ((e >> 7) << 7) + ((e & 63) << 1) + ((e >> 6) & 1)
    for a, table in enumerate((ent_hbm, rel_hbm, ent_hbm, ent_hbm, rel_hbm,
                               ent_hbm)):
      pltpu.make_async_copy(table.at[idx_v.at[buf, a]], row_v.at[buf, a],
                            sem.at[buf]).start()

  def drain(buf):
    for a, table in enumerate((ent_hbm, rel_hbm, ent_hbm, ent_hbm, rel_hbm,
                               ent_hbm)):
      pltpu.make_async_copy(table.at[idx_v.at[buf, a]], row_v.at[buf, a],
                            sem.at[buf]).wait()

  lane = lax.iota(jnp.int32, L)

  def compute(buf, total):
    def body(t, tot):
      dv = jnp.zeros((L,), jnp.float32)
      for k in range(D // L):
        sl = pl.ds(k * L, L)
        dv += jnp.abs(row_v[buf, 0, t, sl] + row_v[buf, 1, t, sl]
                      - row_v[buf, 2, t, sl])
        dv -= jnp.abs(row_v[buf, 3, t, sl] + row_v[buf, 4, t, sl]
                      - row_v[buf, 5, t, sl])
      for shift in (1, 2, 4, 8):
        dv = dv + jnp.take_along_axis(dv, lane ^ shift, axis=0,
                                      mode="promise_in_bounds")
      return tot + jnp.maximum(dv + MARGIN, 0.0)

    return lax.fori_loop(0, CHUNK, body, total, unroll=2)

  total = jnp.zeros((L,), jnp.float32)
  fire(0, 0)
  for c in range(1, NCHUNK):
    fire(c, c % 2)
    drain((c - 1) % 2)
    total = compute((c - 1) % 2, total)
  drain((NCHUNK - 1) % 2)
  total = compute((NCHUNK - 1) % 2, total)

  out_v[...] = jnp.where(lane == 0, total, 0.0)
  pltpu.sync_copy(out_v, out_hbm.at[wid])


@jax.jit
def _transe_call(entity_emb, relation_emb, hp, rp, tp, hn, rn, tn):
  mesh = plsc.VectorSubcoreMesh(
      core_axis_name="c", subcore_axis_name="s", num_cores=NC,
      num_subcores=NS)
  transpose_kernel = pl.pallas_call(
      _tc_transpose_body,
      out_shape=jax.ShapeDtypeStruct((NTC * TCW // 2, 128), jnp.float32),
      grid=(NTC,),
      in_specs=[pl.BlockSpec((D, TCW), lambda i: (0, i))],
      out_specs=pl.BlockSpec((TCW // 2, 128), lambda i: (i, 0)),
  )
  ent_lin = transpose_kernel(entity_emb.T).reshape(NTC * TCW, D)
  grid_kernel = pl.kernel(
      _transe_body,
      out_type=jax.ShapeDtypeStruct((NW, L), jnp.float32),
      mesh=mesh,
      scratch_types=[
          pltpu.VMEM((2, 6, CHUNK), jnp.int32),
          pltpu.VMEM((2, 6, CHUNK, D), jnp.float32),
          pltpu.VMEM((L,), jnp.float32),
          pltpu.SemaphoreType.DMA((2,)),
      ],
      compiler_params=pltpu.CompilerParams(use_tc_tiling_on_sc=False),
  )
  partials = grid_kernel(ent_lin, relation_emb, hp, rp, tp, hn, rn, tn)
  return jnp.sum(partials) / BATCH


def kernel(entity_emb, relation_emb, h_pos, r_pos, t_pos, h_neg, r_neg,
           t_neg):
  shape = (NW, NCHUNK, CHUNK)
  return _transe_call(
      entity_emb, relation_emb,
      h_pos.astype(jnp.int32).reshape(shape),
      r_pos.astype(jnp.int32).reshape(shape),
      t_pos.astype(jnp.int32).reshape(shape),
      h_neg.astype(jnp.int32).reshape(shape),
      r_neg.astype(jnp.int32).reshape(shape),
      t_neg.astype(jnp.int32).reshape(shape),
  )
